# Initial kernel scaffold; baseline (speedup 1.0000x reference)
import jax
import jax.numpy as jnp
from jax.experimental import pallas as pl


def kernel(params, student_id, exercise_id, right_idx, wrong_idx, right_flip_idx, wrong_flip_idx):
    raise NotImplementedError("write your pallas kernel here")


# speedup vs baseline: 2.5398x; 2.5398x over previous
import functools

import jax
import jax.numpy as jnp
from jax import lax
from jax.experimental import pallas as pl
from jax.experimental.pallas import tpu as pltpu
from jax.experimental.pallas import tpu_sc as plsc

_S, _Q, _K = 20000, 10000, 500
_N = _S + _Q + _K
_D = 64
_E = 640000

_NS = 16
_EP = 655360
_ET = _EP // _NS
_BLK = 128
_NBLK = _ET // _BLK
_NP = 30504
_RPT = 1904
_REM = _NP - _NS * _RPT


def _mlp_pallas(x, w1, b1, a1, w2, b2, a2, w3, b3):
    m = x.shape[0]
    bm = 512
    grid = (m + bm - 1) // bm

    def body(x_ref, w1_ref, b1_ref, w2_ref, b2_ref, w3_ref, b3_ref, a_ref, o_ref):
        dn = (((1,), (1,)), ((), ()))
        h = lax.dot_general(x_ref[...], w1_ref[...], dn,
                            preferred_element_type=jnp.float32) + b1_ref[...]
        h = jnp.where(h >= 0, h, a_ref[0] * h)
        h = lax.dot_general(h, w2_ref[...], dn,
                            preferred_element_type=jnp.float32) + b2_ref[...]
        h = jnp.where(h >= 0, h, a_ref[1] * h)
        o_ref[...] = lax.dot_general(h, w3_ref[...], dn,
                                     preferred_element_type=jnp.float32) + b3_ref[...]

    a = jnp.stack([a1, a2]).astype(jnp.float32)
    return pl.pallas_call(
        body,
        grid=(grid,),
        in_specs=[
            pl.BlockSpec((bm, 768), lambda i: (i, 0)),
            pl.BlockSpec((512, 768), lambda i: (0, 0)),
            pl.BlockSpec((1, 512), lambda i: (0, 0)),
            pl.BlockSpec((256, 512), lambda i: (0, 0)),
            pl.BlockSpec((1, 256), lambda i: (0, 0)),
            pl.BlockSpec((64, 256), lambda i: (0, 0)),
            pl.BlockSpec((1, 64), lambda i: (0, 0)),
            pl.BlockSpec(memory_space=pltpu.SMEM),
        ],
        out_specs=pl.BlockSpec((bm, 64), lambda i: (i, 0)),
        out_shape=jax.ShapeDtypeStruct((m, 64), jnp.float32),
    )(x, w1, b1.reshape(1, -1), w2, b2.reshape(1, -1), w3, b3.reshape(1, -1), a)


def _combine_pallas(xcat3, outs3, s, wr, ww, b):
    bm = 4096
    grid = (_N + bm - 1) // bm

    def body(x_ref, r1, r2, r3, w1, w2, w3, wr_ref, ww_ref, b_ref, o_ref):
        dn = (((1,), (1,)), ((), ()))
        rs = r1[0] + r2[0] + r3[0]
        ws = w1[0] + w2[0] + w3[0]
        acc = x_ref[0] + lax.dot_general(rs, wr_ref[...], dn,
                                         preferred_element_type=jnp.float32)
        acc = acc + lax.dot_general(ws, ww_ref[...], dn,
                                    preferred_element_type=jnp.float32)
        o_ref[...] = 0.25 * (acc + 3.0 * b_ref[...])

    def spec(bi):
        return pl.BlockSpec((1, bm, 64), lambda i, b=bi: (b, i, 0))

    in_specs = [spec(s)]
    args = [xcat3]
    for j in range(6):
        in_specs.append(spec(s * 6 + j))
        args.append(outs3)
    in_specs += [
        pl.BlockSpec((64, 64), lambda i: (0, 0)),
        pl.BlockSpec((64, 64), lambda i: (0, 0)),
        pl.BlockSpec((1, 64), lambda i: (0, 0)),
    ]
    args += [wr, ww, b]
    return pl.pallas_call(
        body,
        grid=(grid,),
        in_specs=in_specs,
        out_specs=pl.BlockSpec((bm, 64), lambda i: (i, 0)),
        out_shape=jax.ShapeDtypeStruct((_N, 64), jnp.float32),
    )(*args)


_mesh = plsc.VectorSubcoreMesh(core_axis_name="c", subcore_axis_name="s")


@functools.partial(
    pl.kernel,
    out_type=jax.ShapeDtypeStruct((12 * _NP, _D), jnp.float32),
    mesh=_mesh,
    compiler_params=pltpu.CompilerParams(needs_layout_passes=False,
                                         use_tc_tiling_on_sc=False),
    scratch_types=[
        pltpu.VMEM_SHARED((_NP, _D), jnp.float32),
        pltpu.VMEM((_BLK,), jnp.int32),
        pltpu.VMEM((_BLK,), jnp.int32),
        pltpu.VMEM((_BLK,), jnp.float32),
        pltpu.VMEM((_BLK, _D), jnp.float32),
        pltpu.SemaphoreType.DMA,
        pltpu.SemaphoreType.DMA,
    ],
)
def _propagate(xcat, zeros, cols_r, rows_r, vals_r, cols_w, rows_w, vals_w,
               outs, acc, cols_v, rows_v, vals_v, rowbuf, gsem, ssem):
    cid = lax.axis_index("c")
    sid = lax.axis_index("s")
    graphs = ((cols_r, rows_r, vals_r), (cols_w, rows_w, vals_w))
    for chain in range(2):
        colsg, rowsg, valsg = graphs[chain]
        for k in range(3):
            oidx = chain * 3 + k
            tab = xcat if k == 0 else outs
            off = cid * _N if k == 0 else (cid * 6 + oidx - 1) * _NP
            plsc.subcore_barrier()
            pltpu.sync_copy(zeros.at[pl.ds(sid * _RPT, _RPT)],
                            acc.at[pl.ds(sid * _RPT, _RPT)])

            @pl.when(sid == _NS - 1)
            def _zero_rem():
                pltpu.sync_copy(zeros.at[pl.ds(_NS * _RPT, _REM)],
                                acc.at[pl.ds(_NS * _RPT, _REM)])

            plsc.subcore_barrier()

            offv = jnp.full((16,), off, jnp.int32)

            def blk_body(blk, carry, colsg=colsg, rowsg=rowsg, valsg=valsg,
                         tab=tab, offv=offv):
                ebase = sid * _ET + blk * _BLK
                pltpu.sync_copy(colsg.at[pl.ds(ebase, _BLK)], cols_v)
                pltpu.sync_copy(rowsg.at[pl.ds(ebase, _BLK)], rows_v)
                pltpu.sync_copy(valsg.at[pl.ds(ebase, _BLK)], vals_v)
                for c in range(8):
                    cols_v[pl.ds(c * 16, 16)] = (
                        cols_v[pl.ds(c * 16, 16)] + offv)
                pltpu.async_copy(tab.at[cols_v], rowbuf, gsem).wait()

                def grp(g, c2):
                    for e in range(16):
                        r = g * 16 + e
                        sv = plsc.load_gather(
                            vals_v, [jnp.full((16,), r, jnp.int32)])
                        for c in range(4):
                            rowbuf[r, pl.ds(c * 16, 16)] = (
                                rowbuf[r, pl.ds(c * 16, 16)] * sv)
                    return c2

                lax.fori_loop(0, _BLK // 16, grp, 0)
                pltpu.async_copy(rowbuf, acc.at[rows_v], ssem,
                                 add=True).wait()
                return carry

            lax.fori_loop(0, _NBLK, blk_body, 0)
            plsc.subcore_barrier()
            obase = (cid * 6 + oidx) * _NP
            pltpu.sync_copy(acc.at[pl.ds(sid * _RPT, _RPT)],
                            outs.at[pl.ds(obase + sid * _RPT, _RPT)])

            @pl.when(sid == _NS - 1)
            def _drain_rem():
                pltpu.sync_copy(acc.at[pl.ds(_NS * _RPT, _REM)],
                                outs.at[pl.ds(obase + _NS * _RPT, _REM)])


def _prep_edges(idx, val):
    pad = _EP - _E
    cols = jnp.pad(idx[1].astype(jnp.int32), (0, pad))
    rows = jnp.pad(idx[0].astype(jnp.int32), (0, pad))
    val = jnp.pad(val, (0, pad))
    return cols, rows, val


def kernel(params, student_id, exercise_id, right_idx, wrong_idx,
           right_flip_idx, wrong_flip_idx):
    p = params
    m_stu = _mlp_pallas(p['emb_student'], p['stu_W1'], p['stu_b1'], p['stu_a1'],
                        p['stu_W2'], p['stu_b2'], p['stu_a2'],
                        p['stu_W3'], p['stu_b3'])
    m_exer = _mlp_pallas(p['emb_exercise'], p['exer_W1'], p['exer_b1'],
                         p['exer_a1'], p['exer_W2'], p['exer_b2'],
                         p['exer_a2'], p['exer_W3'], p['exer_b3'])
    m_know = _mlp_pallas(p['emb_knowledge'], p['know_W1'], p['know_b1'],
                         p['know_a1'], p['know_W2'], p['know_b2'],
                         p['know_a2'], p['know_W3'], p['know_b3'])
    x1 = jnp.concatenate([m_stu, m_exer, m_know], axis=0)
    x2 = jnp.concatenate([p['stu_table'], p['exer_table'], p['know_param']],
                         axis=0)
    xcat = jnp.concatenate([x1, x2], axis=0)
    zeros = jnp.zeros((_NP, _D), jnp.float32)
    cr, rr, vr = _prep_edges(right_idx, p['right_val'])
    cw, rw, vw = _prep_edges(wrong_idx, p['wrong_val'])
    outs = _propagate(xcat, zeros, cr, rr, vr, cw, rw, vw)
    outs3 = outs.reshape(12, _NP, _D)
    xcat3 = xcat.reshape(2, _N, _D)
    wr = p['concat_W'][:, :_D]
    ww = p['concat_W'][:, _D:]
    b = p['concat_b'].reshape(1, _D)
    out1 = _combine_pallas(xcat3, outs3, 0, wr, ww, b)
    out2 = _combine_pallas(xcat3, outs3, 1, wr, ww, b)
    s1 = jnp.take(out1, student_id, axis=0)
    d1 = jnp.take(out1, _S + exercise_id, axis=0)
    s2 = jnp.take(out2, student_id, axis=0)
    d2 = jnp.take(out2, _S + exercise_id, axis=0)
    return s1, d1, s2, d2, out1[_S + _Q:], out2[_S + _Q:]


# speedup vs baseline: 3.6537x; 1.4386x over previous
import functools

import jax
import jax.numpy as jnp
from jax import lax
from jax.experimental import pallas as pl
from jax.experimental.pallas import tpu as pltpu
from jax.experimental.pallas import tpu_sc as plsc

_S, _Q, _K = 20000, 10000, 500
_N = _S + _Q + _K
_D = 64
_E = 640000

_NS = 16
_BLK = 64
_NBR = 625
_NBP = _NBR + 3
_NBA = _NBP + 1
_NP = 30504
_RPT = 1904
_REM = _NP - _NS * _RPT


def _mlp_pallas(x, w1, b1, a1, w2, b2, a2, w3, b3):
    m = x.shape[0]
    bm = 512
    grid = (m + bm - 1) // bm

    def body(x_ref, w1_ref, b1_ref, w2_ref, b2_ref, w3_ref, b3_ref, a_ref, o_ref):
        dn = (((1,), (1,)), ((), ()))
        h = lax.dot_general(x_ref[...], w1_ref[...], dn,
                            preferred_element_type=jnp.float32) + b1_ref[...]
        h = jnp.where(h >= 0, h, a_ref[0] * h)
        h = lax.dot_general(h, w2_ref[...], dn,
                            preferred_element_type=jnp.float32) + b2_ref[...]
        h = jnp.where(h >= 0, h, a_ref[1] * h)
        o_ref[...] = lax.dot_general(h, w3_ref[...], dn,
                                     preferred_element_type=jnp.float32) + b3_ref[...]

    a = jnp.stack([a1, a2]).astype(jnp.float32)
    return pl.pallas_call(
        body,
        grid=(grid,),
        in_specs=[
            pl.BlockSpec((bm, 768), lambda i: (i, 0)),
            pl.BlockSpec((512, 768), lambda i: (0, 0)),
            pl.BlockSpec((1, 512), lambda i: (0, 0)),
            pl.BlockSpec((256, 512), lambda i: (0, 0)),
            pl.BlockSpec((1, 256), lambda i: (0, 0)),
            pl.BlockSpec((64, 256), lambda i: (0, 0)),
            pl.BlockSpec((1, 64), lambda i: (0, 0)),
            pl.BlockSpec(memory_space=pltpu.SMEM),
        ],
        out_specs=pl.BlockSpec((bm, 64), lambda i: (i, 0)),
        out_shape=jax.ShapeDtypeStruct((m, 64), jnp.float32),
    )(x, w1, b1.reshape(1, -1), w2, b2.reshape(1, -1), w3, b3.reshape(1, -1), a)


def _combine_pallas(xcat3, outs3, s, wr, ww, b):
    bm = 4096
    grid = (_N + bm - 1) // bm

    def body(x_ref, r1, r2, r3, w1, w2, w3, wr_ref, ww_ref, b_ref, o_ref):
        dn = (((1,), (1,)), ((), ()))
        rs = r1[0] + r2[0] + r3[0]
        ws = w1[0] + w2[0] + w3[0]
        acc = x_ref[0] + lax.dot_general(rs, wr_ref[...], dn,
                                         preferred_element_type=jnp.float32)
        acc = acc + lax.dot_general(ws, ww_ref[...], dn,
                                    preferred_element_type=jnp.float32)
        o_ref[...] = 0.25 * (acc + 3.0 * b_ref[...])

    def spec(bi):
        return pl.BlockSpec((1, bm, 64), lambda i, b=bi: (b, i, 0))

    in_specs = [spec(s)]
    args = [xcat3]
    for j in range(6):
        in_specs.append(spec(s * 6 + j))
        args.append(outs3)
    in_specs += [
        pl.BlockSpec((64, 64), lambda i: (0, 0)),
        pl.BlockSpec((64, 64), lambda i: (0, 0)),
        pl.BlockSpec((1, 64), lambda i: (0, 0)),
    ]
    args += [wr, ww, b]
    return pl.pallas_call(
        body,
        grid=(grid,),
        in_specs=in_specs,
        out_specs=pl.BlockSpec((bm, 64), lambda i: (i, 0)),
        out_shape=jax.ShapeDtypeStruct((_N, 64), jnp.float32),
    )(*args)


_mesh = plsc.VectorSubcoreMesh(core_axis_name="c", subcore_axis_name="s")


@functools.partial(
    pl.kernel,
    out_type=jax.ShapeDtypeStruct((12 * _NP, _D), jnp.float32),
    mesh=_mesh,
    compiler_params=pltpu.CompilerParams(needs_layout_passes=False,
                                         use_tc_tiling_on_sc=False),
    scratch_types=[
        pltpu.VMEM_SHARED((_NP, _D), jnp.float32),
        pltpu.VMEM((3, _BLK), jnp.int32),
        pltpu.VMEM((3, _BLK), jnp.int32),
        pltpu.VMEM((_BLK,), jnp.int32),
        pltpu.VMEM((_BLK,), jnp.int32),
        pltpu.VMEM((_BLK, _D), jnp.float32),
        pltpu.VMEM((_BLK, _D), jnp.float32),
        pltpu.SemaphoreType.DMA,
        pltpu.SemaphoreType.DMA,
        pltpu.SemaphoreType.DMA,
        pltpu.SemaphoreType.DMA,
        pltpu.SemaphoreType.DMA,
        pltpu.SemaphoreType.DMA,
    ],
)
def _propagate(xcat, zeros, pack_r, pack_w, outs, acc,
               ib0, ib1, sb0, sb1, rb0, rb1,
               isem0, isem1, gsem0, gsem1, csem0, csem1):
    cid = lax.axis_index("c")
    sid = lax.axis_index("s")
    ibs, sbs, rbs = (ib0, ib1), (sb0, sb1), (rb0, rb1)
    isems, gsems, csems = (isem0, isem1), (gsem0, gsem1), (csem0, csem1)
    base = sid * _NBA
    graphs = (pack_r, pack_w)
    for chain in range(2):
        packed = graphs[chain]
        for k in range(3):
            oidx = chain * 3 + k
            tab = xcat if k == 0 else outs
            off = cid * _N if k == 0 else (cid * 6 + oidx - 1) * _NP
            plsc.subcore_barrier()
            pltpu.sync_copy(zeros.at[pl.ds(sid * _RPT, _RPT)],
                            acc.at[pl.ds(sid * _RPT, _RPT)])

            @pl.when(sid == _NS - 1)
            def _zero_rem():
                pltpu.sync_copy(zeros.at[pl.ds(_NS * _RPT, _REM)],
                                acc.at[pl.ds(_NS * _RPT, _REM)])

            plsc.subcore_barrier()

            offv = jnp.full((16,), off, jnp.int32)
            two = jnp.full((16,), 2, jnp.int32)

            def idx_issue(j, b):
                pltpu.async_copy(packed.at[base + j], ibs[b], isems[b])

            def idx_wait(b):
                pltpu.make_async_copy(packed.at[base], ibs[b],
                                      isems[b]).wait()

            def offset_add(b):
                for c in range(4):
                    ibs[b][0, pl.ds(c * 16, 16)] = (
                        ibs[b][0, pl.ds(c * 16, 16)] + offv)

            def g_issue(b, tab=tab):
                pltpu.async_copy(tab.at[ibs[b].at[0]], rbs[b], gsems[b])

            def g_wait(b, tab=tab):
                pltpu.make_async_copy(tab.at[ibs[b].at[0]], rbs[b],
                                      gsems[b]).wait()

            def c_issue(b):
                pltpu.async_copy(rbs[b], acc.at[sbs[b]], csems[b],
                                 add=True)

            def c_wait(b):
                pltpu.make_async_copy(rbs[b], acc.at[sbs[b]],
                                      csems[b]).wait()

            def scale(b):
                rb = rbs[b]
                ib = ibs[b]

                def grp(g, c2):
                    for e in range(16):
                        r = g * 16 + e
                        svi = plsc.load_gather(
                            ib, [two, jnp.full((16,), r, jnp.int32)])
                        sv = plsc.bitcast(svi, jnp.float32)
                        for c in range(4):
                            rb[r, pl.ds(c * 16, 16)] = (
                                rb[r, pl.ds(c * 16, 16)] * sv)
                    return c2

                lax.fori_loop(0, _BLK // 16, grp, 0)
                for c in range(4):
                    sbs[b][pl.ds(c * 16, 16)] = ib[1, pl.ds(c * 16, 16)]

            idx_issue(0, 0)
            idx_wait(0)
            offset_add(0)
            g_issue(0)
            idx_issue(1, 1)

            def pair(g2, carry):
                for b in (0, 1):
                    j = g2 * 2 + b
                    nb = 1 - b
                    g_wait(b)
                    scale(b)
                    c_issue(b)
                    idx_wait(nb)
                    offset_add(nb)

                    @pl.when(j > 0)
                    def _():
                        c_wait(nb)

                    @pl.when(j <= _NBP - 2)
                    def _():
                        g_issue(nb)
                        idx_issue(j + 2, b)
                return carry

            lax.fori_loop(0, _NBP // 2, pair, 0)
            c_wait(1)
            plsc.subcore_barrier()
            obase = (cid * 6 + oidx) * _NP
            pltpu.sync_copy(acc.at[pl.ds(sid * _RPT, _RPT)],
                            outs.at[pl.ds(obase + sid * _RPT, _RPT)])

            @pl.when(sid == _NS - 1)
            def _drain_rem():
                pltpu.sync_copy(acc.at[pl.ds(_NS * _RPT, _REM)],
                                outs.at[pl.ds(obase + _NS * _RPT, _REM)])


def _prep_edges(idx, val):
    cols = idx[1].astype(jnp.int32).reshape(_NS, _NBR, _BLK)
    rows = idx[0].astype(jnp.int32).reshape(_NS, _NBR, _BLK)
    vi = lax.bitcast_convert_type(val, jnp.int32).reshape(_NS, _NBR, _BLK)
    rec = jnp.stack([cols, rows, vi], axis=2)
    rec = jnp.pad(rec, ((0, 0), (0, _NBA - _NBR), (0, 0), (0, 0)))
    return rec.reshape(_NS * _NBA, 3, _BLK)


def kernel(params, student_id, exercise_id, right_idx, wrong_idx,
           right_flip_idx, wrong_flip_idx):
    p = params
    m_stu = _mlp_pallas(p['emb_student'], p['stu_W1'], p['stu_b1'], p['stu_a1'],
                        p['stu_W2'], p['stu_b2'], p['stu_a2'],
                        p['stu_W3'], p['stu_b3'])
    m_exer = _mlp_pallas(p['emb_exercise'], p['exer_W1'], p['exer_b1'],
                         p['exer_a1'], p['exer_W2'], p['exer_b2'],
                         p['exer_a2'], p['exer_W3'], p['exer_b3'])
    m_know = _mlp_pallas(p['emb_knowledge'], p['know_W1'], p['know_b1'],
                         p['know_a1'], p['know_W2'], p['know_b2'],
                         p['know_a2'], p['know_W3'], p['know_b3'])
    x1 = jnp.concatenate([m_stu, m_exer, m_know], axis=0)
    x2 = jnp.concatenate([p['stu_table'], p['exer_table'], p['know_param']],
                         axis=0)
    xcat = jnp.concatenate([x1, x2], axis=0)
    zeros = jnp.zeros((_NP, _D), jnp.float32)
    pack_r = _prep_edges(right_idx, p['right_val'])
    pack_w = _prep_edges(wrong_idx, p['wrong_val'])
    outs = _propagate(xcat, zeros, pack_r, pack_w)
    outs3 = outs.reshape(12, _NP, _D)
    xcat3 = xcat.reshape(2, _N, _D)
    wr = p['concat_W'][:, :_D]
    ww = p['concat_W'][:, _D:]
    b = p['concat_b'].reshape(1, _D)
    out1 = _combine_pallas(xcat3, outs3, 0, wr, ww, b)
    out2 = _combine_pallas(xcat3, outs3, 1, wr, ww, b)
    s1 = jnp.take(out1, student_id, axis=0)
    d1 = jnp.take(out1, _S + exercise_id, axis=0)
    s2 = jnp.take(out2, student_id, axis=0)
    d2 = jnp.take(out2, _S + exercise_id, axis=0)
    return s1, d1, s2, d2, out1[_S + _Q:], out2[_S + _Q:]


# speedup vs baseline: 4.2051x; 1.1509x over previous
import functools

import jax
import jax.numpy as jnp
from jax import lax
from jax.experimental import pallas as pl
from jax.experimental.pallas import tpu as pltpu
from jax.experimental.pallas import tpu_sc as plsc

_S, _Q, _K = 20000, 10000, 500
_N = _S + _Q + _K
_D = 64
_E = 640000

_NS = 16
_BLK = 64
_NBR = 625
_NBP = _NBR + 3
_NBA = _NBP + 1
_NP = 30504
_RPT = 1904
_REM = _NP - _NS * _RPT


def _mlp_pallas(x, w1, b1, a1, w2, b2, a2, w3, b3):
    m = x.shape[0]
    bm = 512
    grid = (m + bm - 1) // bm

    def body(x_ref, w1_ref, b1_ref, w2_ref, b2_ref, w3_ref, b3_ref, a_ref, o_ref):
        dn = (((1,), (1,)), ((), ()))
        h = lax.dot_general(x_ref[...], w1_ref[...], dn,
                            preferred_element_type=jnp.float32) + b1_ref[...]
        h = jnp.where(h >= 0, h, a_ref[0] * h)
        h = lax.dot_general(h, w2_ref[...], dn,
                            preferred_element_type=jnp.float32) + b2_ref[...]
        h = jnp.where(h >= 0, h, a_ref[1] * h)
        o_ref[...] = lax.dot_general(h, w3_ref[...], dn,
                                     preferred_element_type=jnp.float32) + b3_ref[...]

    a = jnp.stack([a1, a2]).astype(jnp.float32)
    return pl.pallas_call(
        body,
        grid=(grid,),
        in_specs=[
            pl.BlockSpec((bm, 768), lambda i: (i, 0)),
            pl.BlockSpec((512, 768), lambda i: (0, 0)),
            pl.BlockSpec((1, 512), lambda i: (0, 0)),
            pl.BlockSpec((256, 512), lambda i: (0, 0)),
            pl.BlockSpec((1, 256), lambda i: (0, 0)),
            pl.BlockSpec((64, 256), lambda i: (0, 0)),
            pl.BlockSpec((1, 64), lambda i: (0, 0)),
            pl.BlockSpec(memory_space=pltpu.SMEM),
        ],
        out_specs=pl.BlockSpec((bm, 64), lambda i: (i, 0)),
        out_shape=jax.ShapeDtypeStruct((m, 64), jnp.float32),
    )(x, w1, b1.reshape(1, -1), w2, b2.reshape(1, -1), w3, b3.reshape(1, -1), a)


def _combine_pallas(xcat3, outs3, s, wr, ww, b):
    bm = 4096
    grid = (_N + bm - 1) // bm

    def body(x_ref, r1, r2, r3, w1, w2, w3, wr_ref, ww_ref, b_ref, o_ref):
        dn = (((1,), (1,)), ((), ()))
        rs = r1[0] + r2[0] + r3[0]
        ws = w1[0] + w2[0] + w3[0]
        acc = x_ref[0] + lax.dot_general(rs, wr_ref[...], dn,
                                         preferred_element_type=jnp.float32)
        acc = acc + lax.dot_general(ws, ww_ref[...], dn,
                                    preferred_element_type=jnp.float32)
        o_ref[...] = 0.25 * (acc + 3.0 * b_ref[...])

    def spec(bi):
        return pl.BlockSpec((1, bm, 64), lambda i, b=bi: (b, i, 0))

    in_specs = [spec(s)]
    args = [xcat3]
    for j in range(6):
        in_specs.append(spec(s * 6 + j))
        args.append(outs3)
    in_specs += [
        pl.BlockSpec((64, 64), lambda i: (0, 0)),
        pl.BlockSpec((64, 64), lambda i: (0, 0)),
        pl.BlockSpec((1, 64), lambda i: (0, 0)),
    ]
    args += [wr, ww, b]
    return pl.pallas_call(
        body,
        grid=(grid,),
        in_specs=in_specs,
        out_specs=pl.BlockSpec((bm, 64), lambda i: (i, 0)),
        out_shape=jax.ShapeDtypeStruct((_N, 64), jnp.float32),
    )(*args)


_mesh = plsc.VectorSubcoreMesh(core_axis_name="c", subcore_axis_name="s")


@functools.partial(
    pl.kernel,
    out_type=jax.ShapeDtypeStruct((12 * _NP, _D), jnp.float32),
    mesh=_mesh,
    compiler_params=pltpu.CompilerParams(needs_layout_passes=False,
                                         use_tc_tiling_on_sc=False),
    scratch_types=[
        pltpu.VMEM_SHARED((_NP, _D), jnp.float32),
        pltpu.VMEM((3, _BLK), jnp.int32),
        pltpu.VMEM((3, _BLK), jnp.int32),
        pltpu.VMEM((_BLK,), jnp.int32),
        pltpu.VMEM((_BLK,), jnp.int32),
        pltpu.VMEM((_BLK, _D), jnp.float32),
        pltpu.VMEM((_BLK, _D), jnp.float32),
        pltpu.SemaphoreType.DMA,
        pltpu.SemaphoreType.DMA,
        pltpu.SemaphoreType.DMA,
        pltpu.SemaphoreType.DMA,
        pltpu.SemaphoreType.DMA,
        pltpu.SemaphoreType.DMA,
    ],
)
def _propagate(xcat, zeros, pack_r, pack_w, outs, acc,
               ib0, ib1, sb0, sb1, rb0, rb1,
               isem0, isem1, gsem0, gsem1, csem0, csem1):
    cid = lax.axis_index("c")
    sid = lax.axis_index("s")
    ibs, sbs, rbs = (ib0, ib1), (sb0, sb1), (rb0, rb1)
    isems, gsems, csems = (isem0, isem1), (gsem0, gsem1), (csem0, csem1)
    base = sid * _NBA
    graphs = (pack_r, pack_w)
    for chain in range(2):
        packed = graphs[chain]
        for k in range(3):
            oidx = chain * 3 + k
            tab = xcat if k == 0 else outs
            off = cid * _N if k == 0 else (cid * 6 + oidx - 1) * _NP
            plsc.subcore_barrier()
            pltpu.sync_copy(zeros.at[pl.ds(sid * _RPT, _RPT)],
                            acc.at[pl.ds(sid * _RPT, _RPT)])

            @pl.when(sid == _NS - 1)
            def _zero_rem():
                pltpu.sync_copy(zeros.at[pl.ds(_NS * _RPT, _REM)],
                                acc.at[pl.ds(_NS * _RPT, _REM)])

            plsc.subcore_barrier()

            offv = jnp.full((16,), off, jnp.int32)
            two = jnp.full((16,), 2, jnp.int32)

            def idx_issue(j, b):
                pltpu.async_copy(packed.at[base + j], ibs[b], isems[b])

            def idx_wait(b):
                pltpu.make_async_copy(packed.at[base], ibs[b],
                                      isems[b]).wait()

            def offset_add(b):
                for c in range(4):
                    ibs[b][0, pl.ds(c * 16, 16)] = (
                        ibs[b][0, pl.ds(c * 16, 16)] + offv)

            def g_issue(b, tab=tab):
                pltpu.async_copy(tab.at[ibs[b].at[0]], rbs[b], gsems[b])

            def g_wait(b, tab=tab):
                pltpu.make_async_copy(tab.at[ibs[b].at[0]], rbs[b],
                                      gsems[b]).wait()

            def c_issue(b):
                pltpu.async_copy(rbs[b], acc.at[sbs[b]], csems[b],
                                 add=True)

            def c_wait(b):
                pltpu.make_async_copy(rbs[b], acc.at[sbs[b]],
                                      csems[b]).wait()

            def scale(b):
                rb = rbs[b]
                ib = ibs[b]

                def grp(g, c2):
                    for e in range(16):
                        r = g * 16 + e
                        svi = plsc.load_gather(
                            ib, [two, jnp.full((16,), r, jnp.int32)])
                        sv = plsc.bitcast(svi, jnp.float32)
                        for c in range(4):
                            rb[r, pl.ds(c * 16, 16)] = (
                                rb[r, pl.ds(c * 16, 16)] * sv)
                    return c2

                lax.fori_loop(0, _BLK // 16, grp, 0)
                for c in range(4):
                    sbs[b][pl.ds(c * 16, 16)] = ib[1, pl.ds(c * 16, 16)]

            idx_issue(0, 0)
            idx_wait(0)
            offset_add(0)
            g_issue(0)
            idx_issue(1, 1)

            def pair(g2, carry):
                for b in (0, 1):
                    j = g2 * 2 + b
                    nb = 1 - b
                    idx_wait(nb)
                    offset_add(nb)

                    @pl.when(j > 0)
                    def _():
                        c_wait(nb)

                    @pl.when(j <= _NBP - 2)
                    def _():
                        g_issue(nb)

                    g_wait(b)
                    scale(b)
                    c_issue(b)

                    @pl.when(j <= _NBP - 2)
                    def _():
                        idx_issue(j + 2, b)
                return carry

            lax.fori_loop(0, _NBP // 2, pair, 0)
            c_wait(1)
            plsc.subcore_barrier()
            obase = (cid * 6 + oidx) * _NP
            pltpu.sync_copy(acc.at[pl.ds(sid * _RPT, _RPT)],
                            outs.at[pl.ds(obase + sid * _RPT, _RPT)])

            @pl.when(sid == _NS - 1)
            def _drain_rem():
                pltpu.sync_copy(acc.at[pl.ds(_NS * _RPT, _REM)],
                                outs.at[pl.ds(obase + _NS * _RPT, _REM)])


def _prep_edges(idx, val):
    cols = idx[1].astype(jnp.int32).reshape(_NS, _NBR, _BLK)
    rows = idx[0].astype(jnp.int32).reshape(_NS, _NBR, _BLK)
    vi = lax.bitcast_convert_type(val, jnp.int32).reshape(_NS, _NBR, _BLK)
    rec = jnp.stack([cols, rows, vi], axis=2)
    rec = jnp.pad(rec, ((0, 0), (0, _NBA - _NBR), (0, 0), (0, 0)))
    return rec.reshape(_NS * _NBA, 3, _BLK)


def kernel(params, student_id, exercise_id, right_idx, wrong_idx,
           right_flip_idx, wrong_flip_idx):
    p = params
    m_stu = _mlp_pallas(p['emb_student'], p['stu_W1'], p['stu_b1'], p['stu_a1'],
                        p['stu_W2'], p['stu_b2'], p['stu_a2'],
                        p['stu_W3'], p['stu_b3'])
    m_exer = _mlp_pallas(p['emb_exercise'], p['exer_W1'], p['exer_b1'],
                         p['exer_a1'], p['exer_W2'], p['exer_b2'],
                         p['exer_a2'], p['exer_W3'], p['exer_b3'])
    m_know = _mlp_pallas(p['emb_knowledge'], p['know_W1'], p['know_b1'],
                         p['know_a1'], p['know_W2'], p['know_b2'],
                         p['know_a2'], p['know_W3'], p['know_b3'])
    x1 = jnp.concatenate([m_stu, m_exer, m_know], axis=0)
    x2 = jnp.concatenate([p['stu_table'], p['exer_table'], p['know_param']],
                         axis=0)
    xcat = jnp.concatenate([x1, x2], axis=0)
    zeros = jnp.zeros((_NP, _D), jnp.float32)
    pack_r = _prep_edges(right_idx, p['right_val'])
    pack_w = _prep_edges(wrong_idx, p['wrong_val'])
    outs = _propagate(xcat, zeros, pack_r, pack_w)
    outs3 = outs.reshape(12, _NP, _D)
    xcat3 = xcat.reshape(2, _N, _D)
    wr = p['concat_W'][:, :_D]
    ww = p['concat_W'][:, _D:]
    b = p['concat_b'].reshape(1, _D)
    out1 = _combine_pallas(xcat3, outs3, 0, wr, ww, b)
    out2 = _combine_pallas(xcat3, outs3, 1, wr, ww, b)
    s1 = jnp.take(out1, student_id, axis=0)
    d1 = jnp.take(out1, _S + exercise_id, axis=0)
    s2 = jnp.take(out2, student_id, axis=0)
    d2 = jnp.take(out2, _S + exercise_id, axis=0)
    return s1, d1, s2, d2, out1[_S + _Q:], out2[_S + _Q:]


# speedup vs baseline: 4.4683x; 1.0626x over previous
import functools

import jax
import jax.numpy as jnp
from jax import lax
from jax.experimental import pallas as pl
from jax.experimental.pallas import tpu as pltpu
from jax.experimental.pallas import tpu_sc as plsc

_S, _Q, _K = 20000, 10000, 500
_N = _S + _Q + _K
_D = 64
_E = 640000

_NS = 16
_BLK = 64
_NBR = 625
_NBP = _NBR + 3
_NBA = _NBP + 1
_NP = 30504
_RPT = 1904
_REM = _NP - _NS * _RPT


def _mlp_pallas(x, w1, b1, a1, w2, b2, a2, w3, b3):
    m = x.shape[0]
    bm = 512
    grid = (m + bm - 1) // bm

    def body(x_ref, w1_ref, b1_ref, w2_ref, b2_ref, w3_ref, b3_ref, a_ref, o_ref):
        dn = (((1,), (1,)), ((), ()))
        h = lax.dot_general(x_ref[...], w1_ref[...], dn,
                            preferred_element_type=jnp.float32) + b1_ref[...]
        h = jnp.where(h >= 0, h, a_ref[0] * h)
        h = lax.dot_general(h, w2_ref[...], dn,
                            preferred_element_type=jnp.float32) + b2_ref[...]
        h = jnp.where(h >= 0, h, a_ref[1] * h)
        o_ref[...] = lax.dot_general(h, w3_ref[...], dn,
                                     preferred_element_type=jnp.float32) + b3_ref[...]

    a = jnp.stack([a1, a2]).astype(jnp.float32)
    return pl.pallas_call(
        body,
        grid=(grid,),
        in_specs=[
            pl.BlockSpec((bm, 768), lambda i: (i, 0)),
            pl.BlockSpec((512, 768), lambda i: (0, 0)),
            pl.BlockSpec((1, 512), lambda i: (0, 0)),
            pl.BlockSpec((256, 512), lambda i: (0, 0)),
            pl.BlockSpec((1, 256), lambda i: (0, 0)),
            pl.BlockSpec((64, 256), lambda i: (0, 0)),
            pl.BlockSpec((1, 64), lambda i: (0, 0)),
            pl.BlockSpec(memory_space=pltpu.SMEM),
        ],
        out_specs=pl.BlockSpec((bm, 64), lambda i: (i, 0)),
        out_shape=jax.ShapeDtypeStruct((m, 64), jnp.float32),
    )(x, w1, b1.reshape(1, -1), w2, b2.reshape(1, -1), w3, b3.reshape(1, -1), a)


def _combine_pallas(xcat3, outs3, s, wr, ww, b):
    bm = 4096
    grid = (_N + bm - 1) // bm

    def body(x_ref, r1, r2, r3, w1, w2, w3, wr_ref, ww_ref, b_ref, o_ref):
        dn = (((1,), (1,)), ((), ()))
        rs = r1[0] + r2[0] + r3[0]
        ws = w1[0] + w2[0] + w3[0]
        acc = x_ref[0] + lax.dot_general(rs, wr_ref[...], dn,
                                         preferred_element_type=jnp.float32)
        acc = acc + lax.dot_general(ws, ww_ref[...], dn,
                                    preferred_element_type=jnp.float32)
        o_ref[...] = 0.25 * (acc + 3.0 * b_ref[...])

    def spec(bi):
        return pl.BlockSpec((1, bm, 64), lambda i, b=bi: (b, i, 0))

    in_specs = [spec(s)]
    args = [xcat3]
    for j in range(6):
        in_specs.append(spec(s * 6 + j))
        args.append(outs3)
    in_specs += [
        pl.BlockSpec((64, 64), lambda i: (0, 0)),
        pl.BlockSpec((64, 64), lambda i: (0, 0)),
        pl.BlockSpec((1, 64), lambda i: (0, 0)),
    ]
    args += [wr, ww, b]
    return pl.pallas_call(
        body,
        grid=(grid,),
        in_specs=in_specs,
        out_specs=pl.BlockSpec((bm, 64), lambda i: (i, 0)),
        out_shape=jax.ShapeDtypeStruct((_N, 64), jnp.float32),
    )(*args)


_mesh = plsc.VectorSubcoreMesh(core_axis_name="c", subcore_axis_name="s")


@functools.partial(
    pl.kernel,
    out_type=jax.ShapeDtypeStruct((12 * _NP, _D), jnp.float32),
    mesh=_mesh,
    compiler_params=pltpu.CompilerParams(needs_layout_passes=False,
                                         use_tc_tiling_on_sc=False),
    scratch_types=[
        pltpu.VMEM_SHARED((_NP, _D), jnp.float32),
        pltpu.VMEM((3, _BLK), jnp.int32),
        pltpu.VMEM((3, _BLK), jnp.int32),
        pltpu.VMEM((_BLK,), jnp.int32),
        pltpu.VMEM((_BLK,), jnp.int32),
        pltpu.VMEM((_BLK, _D), jnp.float32),
        pltpu.VMEM((_BLK, _D), jnp.float32),
        pltpu.SemaphoreType.DMA,
        pltpu.SemaphoreType.DMA,
        pltpu.SemaphoreType.DMA,
        pltpu.SemaphoreType.DMA,
        pltpu.SemaphoreType.DMA,
        pltpu.SemaphoreType.DMA,
    ],
)
def _propagate(xcat, zeros, pack_r, pack_w, outs, acc,
               ib0, ib1, sb0, sb1, rb0, rb1,
               isem0, isem1, gsem0, gsem1, csem0, csem1):
    cid = lax.axis_index("c")
    sid = lax.axis_index("s")
    ibs, sbs, rbs = (ib0, ib1), (sb0, sb1), (rb0, rb1)
    isems, gsems, csems = (isem0, isem1), (gsem0, gsem1), (csem0, csem1)
    base = sid * _NBA
    graphs = (pack_r, pack_w)
    for chain in range(2):
        packed = graphs[chain]
        for k in range(3):
            oidx = chain * 3 + k
            tab = xcat if k == 0 else outs
            off = cid * _N if k == 0 else (cid * 6 + oidx - 1) * _NP
            plsc.subcore_barrier()
            pltpu.sync_copy(zeros.at[pl.ds(sid * _RPT, _RPT)],
                            acc.at[pl.ds(sid * _RPT, _RPT)])

            @pl.when(sid == _NS - 1)
            def _zero_rem():
                pltpu.sync_copy(zeros.at[pl.ds(_NS * _RPT, _REM)],
                                acc.at[pl.ds(_NS * _RPT, _REM)])

            plsc.subcore_barrier()

            offv = jnp.full((16,), off, jnp.int32)

            def idx_issue(j, b):
                pltpu.async_copy(packed.at[base + j], ibs[b], isems[b])

            def idx_wait(b):
                pltpu.make_async_copy(packed.at[base], ibs[b],
                                      isems[b]).wait()

            def offset_add(b):
                for c in range(4):
                    ibs[b][0, pl.ds(c * 16, 16)] = (
                        ibs[b][0, pl.ds(c * 16, 16)] + offv)

            def g_issue(b, tab=tab):
                pltpu.async_copy(tab.at[ibs[b].at[0]], rbs[b], gsems[b])

            def g_wait(b, tab=tab):
                pltpu.make_async_copy(tab.at[ibs[b].at[0]], rbs[b],
                                      gsems[b]).wait()

            def c_issue(b):
                pltpu.async_copy(rbs[b], acc.at[sbs[b]], csems[b],
                                 add=True)

            def c_wait(b):
                pltpu.make_async_copy(rbs[b], acc.at[sbs[b]],
                                      csems[b]).wait()

            def scale(b):
                rb = rbs[b]
                ib = ibs[b]

                def grp(g, c2):
                    vv = plsc.bitcast(ib[2, pl.ds(g * 16, 16)], jnp.float32)
                    for e in range(16):
                        r = g * 16 + e
                        sv = vv.at[jnp.full((16,), e, jnp.int32)].get(
                            mode='promise_in_bounds')
                        for c in range(4):
                            rb[r, pl.ds(c * 16, 16)] = (
                                rb[r, pl.ds(c * 16, 16)] * sv)
                    return c2

                lax.fori_loop(0, _BLK // 16, grp, 0)
                for c in range(4):
                    sbs[b][pl.ds(c * 16, 16)] = ib[1, pl.ds(c * 16, 16)]

            idx_issue(0, 0)
            idx_wait(0)
            offset_add(0)
            g_issue(0)
            idx_issue(1, 1)

            def pair(g2, carry):
                for b in (0, 1):
                    j = g2 * 2 + b
                    nb = 1 - b
                    idx_wait(nb)
                    offset_add(nb)

                    @pl.when(j > 0)
                    def _():
                        c_wait(nb)

                    @pl.when(j <= _NBP - 2)
                    def _():
                        g_issue(nb)

                    g_wait(b)
                    scale(b)
                    c_issue(b)

                    @pl.when(j <= _NBP - 2)
                    def _():
                        idx_issue(j + 2, b)
                return carry

            lax.fori_loop(0, _NBP // 2, pair, 0)
            c_wait(1)
            plsc.subcore_barrier()
            obase = (cid * 6 + oidx) * _NP
            pltpu.sync_copy(acc.at[pl.ds(sid * _RPT, _RPT)],
                            outs.at[pl.ds(obase + sid * _RPT, _RPT)])

            @pl.when(sid == _NS - 1)
            def _drain_rem():
                pltpu.sync_copy(acc.at[pl.ds(_NS * _RPT, _REM)],
                                outs.at[pl.ds(obase + _NS * _RPT, _REM)])


def _prep_edges(idx, val):
    cols = idx[1].astype(jnp.int32).reshape(_NS, _NBR, _BLK)
    rows = idx[0].astype(jnp.int32).reshape(_NS, _NBR, _BLK)
    vi = lax.bitcast_convert_type(val, jnp.int32).reshape(_NS, _NBR, _BLK)
    rec = jnp.stack([cols, rows, vi], axis=2)
    rec = jnp.pad(rec, ((0, 0), (0, _NBA - _NBR), (0, 0), (0, 0)))
    return rec.reshape(_NS * _NBA, 3, _BLK)


def kernel(params, student_id, exercise_id, right_idx, wrong_idx,
           right_flip_idx, wrong_flip_idx):
    p = params
    m_stu = _mlp_pallas(p['emb_student'], p['stu_W1'], p['stu_b1'], p['stu_a1'],
                        p['stu_W2'], p['stu_b2'], p['stu_a2'],
                        p['stu_W3'], p['stu_b3'])
    m_exer = _mlp_pallas(p['emb_exercise'], p['exer_W1'], p['exer_b1'],
                         p['exer_a1'], p['exer_W2'], p['exer_b2'],
                         p['exer_a2'], p['exer_W3'], p['exer_b3'])
    m_know = _mlp_pallas(p['emb_knowledge'], p['know_W1'], p['know_b1'],
                         p['know_a1'], p['know_W2'], p['know_b2'],
                         p['know_a2'], p['know_W3'], p['know_b3'])
    x1 = jnp.concatenate([m_stu, m_exer, m_know], axis=0)
    x2 = jnp.concatenate([p['stu_table'], p['exer_table'], p['know_param']],
                         axis=0)
    xcat = jnp.concatenate([x1, x2], axis=0)
    zeros = jnp.zeros((_NP, _D), jnp.float32)
    pack_r = _prep_edges(right_idx, p['right_val'])
    pack_w = _prep_edges(wrong_idx, p['wrong_val'])
    outs = _propagate(xcat, zeros, pack_r, pack_w)
    outs3 = outs.reshape(12, _NP, _D)
    xcat3 = xcat.reshape(2, _N, _D)
    wr = p['concat_W'][:, :_D]
    ww = p['concat_W'][:, _D:]
    b = p['concat_b'].reshape(1, _D)
    out1 = _combine_pallas(xcat3, outs3, 0, wr, ww, b)
    out2 = _combine_pallas(xcat3, outs3, 1, wr, ww, b)
    s1 = jnp.take(out1, student_id, axis=0)
    d1 = jnp.take(out1, _S + exercise_id, axis=0)
    s2 = jnp.take(out2, student_id, axis=0)
    d2 = jnp.take(out2, _S + exercise_id, axis=0)
    return s1, d1, s2, d2, out1[_S + _Q:], out2[_S + _Q:]


# speedup vs baseline: 7.8483x; 1.7564x over previous
import functools

import jax
import jax.numpy as jnp
from jax import lax
from jax.experimental import pallas as pl
from jax.experimental.pallas import tpu as pltpu
from jax.experimental.pallas import tpu_sc as plsc

_S, _Q, _K = 20000, 10000, 500
_N = _S + _Q + _K
_D = 64
_E = 640000

_NS = 16
_BLK = 64
_NBR = 625
_NBP = _NBR + 3
_NBA = _NBP + 1
_NP = 30504
_RPT = 1904
_REM = _NP - _NS * _RPT


def _mlp_pallas(x, w1, b1, a1, w2, b2, a2, w3, b3):
    m = x.shape[0]
    bm = 512
    grid = (m + bm - 1) // bm

    def body(x_ref, w1_ref, b1_ref, w2_ref, b2_ref, w3_ref, b3_ref, a_ref, o_ref):
        dn = (((1,), (1,)), ((), ()))
        h = lax.dot_general(x_ref[...], w1_ref[...], dn,
                            preferred_element_type=jnp.float32) + b1_ref[...]
        h = jnp.where(h >= 0, h, a_ref[0] * h)
        h = lax.dot_general(h, w2_ref[...], dn,
                            preferred_element_type=jnp.float32) + b2_ref[...]
        h = jnp.where(h >= 0, h, a_ref[1] * h)
        o_ref[...] = lax.dot_general(h, w3_ref[...], dn,
                                     preferred_element_type=jnp.float32) + b3_ref[...]

    a = jnp.stack([a1, a2]).astype(jnp.float32)
    return pl.pallas_call(
        body,
        grid=(grid,),
        in_specs=[
            pl.BlockSpec((bm, 768), lambda i: (i, 0)),
            pl.BlockSpec((512, 768), lambda i: (0, 0)),
            pl.BlockSpec((1, 512), lambda i: (0, 0)),
            pl.BlockSpec((256, 512), lambda i: (0, 0)),
            pl.BlockSpec((1, 256), lambda i: (0, 0)),
            pl.BlockSpec((64, 256), lambda i: (0, 0)),
            pl.BlockSpec((1, 64), lambda i: (0, 0)),
            pl.BlockSpec(memory_space=pltpu.SMEM),
        ],
        out_specs=pl.BlockSpec((bm, 64), lambda i: (i, 0)),
        out_shape=jax.ShapeDtypeStruct((m, 64), jnp.float32),
    )(x, w1, b1.reshape(1, -1), w2, b2.reshape(1, -1), w3, b3.reshape(1, -1), a)


def _combine_pallas(xcat3, outs3, s, wr, ww, b):
    bm = 4096
    grid = (_N + bm - 1) // bm

    def body(x_ref, r1, r2, r3, w1, w2, w3, wr_ref, ww_ref, b_ref, o_ref):
        dn = (((1,), (1,)), ((), ()))
        rs = r1[0] + r2[0] + r3[0]
        ws = w1[0] + w2[0] + w3[0]
        acc = x_ref[0] + lax.dot_general(rs, wr_ref[...], dn,
                                         preferred_element_type=jnp.float32)
        acc = acc + lax.dot_general(ws, ww_ref[...], dn,
                                    preferred_element_type=jnp.float32)
        o_ref[...] = 0.25 * (acc + 3.0 * b_ref[...])

    def spec(bi):
        return pl.BlockSpec((1, bm, 64), lambda i, b=bi: (b, i, 0))

    in_specs = [spec(s)]
    args = [xcat3]
    for j in range(6):
        in_specs.append(spec(s * 6 + j))
        args.append(outs3)
    in_specs += [
        pl.BlockSpec((64, 64), lambda i: (0, 0)),
        pl.BlockSpec((64, 64), lambda i: (0, 0)),
        pl.BlockSpec((1, 64), lambda i: (0, 0)),
    ]
    args += [wr, ww, b]
    return pl.pallas_call(
        body,
        grid=(grid,),
        in_specs=in_specs,
        out_specs=pl.BlockSpec((bm, 64), lambda i: (i, 0)),
        out_shape=jax.ShapeDtypeStruct((_N, 64), jnp.float32),
    )(*args)


_mesh = plsc.VectorSubcoreMesh(core_axis_name="c", subcore_axis_name="s")


@functools.partial(
    pl.kernel,
    out_type=jax.ShapeDtypeStruct((12 * _NP, _D), jnp.float32),
    mesh=_mesh,
    compiler_params=pltpu.CompilerParams(needs_layout_passes=False,
                                         use_tc_tiling_on_sc=False),
    scratch_types=[
        pltpu.VMEM_SHARED((_NP, _D), jnp.float32),
        pltpu.VMEM((3, _BLK), jnp.int32),
        pltpu.VMEM((3, _BLK), jnp.int32),
        pltpu.VMEM((_BLK,), jnp.int32),
        pltpu.VMEM((_BLK,), jnp.int32),
        pltpu.VMEM((_BLK, _D), jnp.float32),
        pltpu.VMEM((_BLK, _D), jnp.float32),
        pltpu.SemaphoreType.DMA,
        pltpu.SemaphoreType.DMA,
        pltpu.SemaphoreType.DMA,
        pltpu.SemaphoreType.DMA,
        pltpu.SemaphoreType.DMA,
        pltpu.SemaphoreType.DMA,
    ],
)
def _propagate(xcat, zeros, pack_r, pack_w, outs, acc,
               ib0, ib1, sb0, sb1, rb0, rb1,
               isem0, isem1, gsem0, gsem1, csem0, csem1):
    cid = lax.axis_index("c")
    sid = lax.axis_index("s")
    ibs, sbs, rbs = (ib0, ib1), (sb0, sb1), (rb0, rb1)
    isems, gsems, csems = (isem0, isem1), (gsem0, gsem1), (csem0, csem1)
    base = sid * _NBA
    graphs = (pack_r, pack_w)
    for chain in range(2):
        packed = graphs[chain]
        for k in range(3):
            oidx = chain * 3 + k
            tab = xcat if k == 0 else outs
            off = cid * _N if k == 0 else (cid * 6 + oidx - 1) * _NP
            plsc.subcore_barrier()
            pltpu.sync_copy(zeros.at[pl.ds(sid * _RPT, _RPT)],
                            acc.at[pl.ds(sid * _RPT, _RPT)])

            @pl.when(sid == _NS - 1)
            def _zero_rem():
                pltpu.sync_copy(zeros.at[pl.ds(_NS * _RPT, _REM)],
                                acc.at[pl.ds(_NS * _RPT, _REM)])

            plsc.subcore_barrier()

            offv = jnp.full((16,), off, jnp.int32)

            def idx_issue(j, b):
                pltpu.async_copy(packed.at[base + j], ibs[b], isems[b])

            def idx_wait(b):
                pltpu.make_async_copy(packed.at[base], ibs[b],
                                      isems[b]).wait()

            def offset_add(b):
                for c in range(4):
                    ibs[b][0, pl.ds(c * 16, 16)] = (
                        ibs[b][0, pl.ds(c * 16, 16)] + offv)

            def g_issue(b, tab=tab):
                pltpu.async_copy(tab.at[ibs[b].at[0]], rbs[b], gsems[b])

            def g_wait(b, tab=tab):
                pltpu.make_async_copy(tab.at[ibs[b].at[0]], rbs[b],
                                      gsems[b]).wait()

            def c_issue(b):
                pltpu.async_copy(rbs[b], acc.at[sbs[b]], csems[b],
                                 add=True)

            def c_wait(b):
                pltpu.make_async_copy(rbs[b], acc.at[sbs[b]],
                                      csems[b]).wait()

            def scale(b):
                rb = rbs[b]
                ib = ibs[b]

                for g in range(_BLK // 16):
                    vv = plsc.bitcast(ib[2, pl.ds(g * 16, 16)], jnp.float32)
                    for e in range(16):
                        r = g * 16 + e
                        sv = vv.at[jnp.full((16,), e, jnp.int32)].get(
                            mode='promise_in_bounds')
                        for c in range(4):
                            rb[r, pl.ds(c * 16, 16)] = (
                                rb[r, pl.ds(c * 16, 16)] * sv)
                for c in range(4):
                    sbs[b][pl.ds(c * 16, 16)] = ib[1, pl.ds(c * 16, 16)]

            idx_issue(0, 0)
            idx_wait(0)
            offset_add(0)
            g_issue(0)
            idx_issue(1, 1)

            def pair(g2, carry):
                for b in (0, 1):
                    j = g2 * 2 + b
                    nb = 1 - b
                    idx_wait(nb)
                    offset_add(nb)

                    @pl.when(j > 0)
                    def _():
                        c_wait(nb)

                    @pl.when(j <= _NBP - 2)
                    def _():
                        g_issue(nb)

                    g_wait(b)
                    scale(b)
                    c_issue(b)

                    @pl.when(j <= _NBP - 2)
                    def _():
                        idx_issue(j + 2, b)
                return carry

            lax.fori_loop(0, _NBP // 2, pair, 0)
            c_wait(1)
            plsc.subcore_barrier()
            obase = (cid * 6 + oidx) * _NP
            pltpu.sync_copy(acc.at[pl.ds(sid * _RPT, _RPT)],
                            outs.at[pl.ds(obase + sid * _RPT, _RPT)])

            @pl.when(sid == _NS - 1)
            def _drain_rem():
                pltpu.sync_copy(acc.at[pl.ds(_NS * _RPT, _REM)],
                                outs.at[pl.ds(obase + _NS * _RPT, _REM)])


def _prep_edges(idx, val):
    cols = idx[1].astype(jnp.int32).reshape(_NS, _NBR, _BLK)
    rows = idx[0].astype(jnp.int32).reshape(_NS, _NBR, _BLK)
    vi = lax.bitcast_convert_type(val, jnp.int32).reshape(_NS, _NBR, _BLK)
    rec = jnp.stack([cols, rows, vi], axis=2)
    rec = jnp.pad(rec, ((0, 0), (0, _NBA - _NBR), (0, 0), (0, 0)))
    return rec.reshape(_NS * _NBA, 3, _BLK)


def kernel(params, student_id, exercise_id, right_idx, wrong_idx,
           right_flip_idx, wrong_flip_idx):
    p = params
    m_stu = _mlp_pallas(p['emb_student'], p['stu_W1'], p['stu_b1'], p['stu_a1'],
                        p['stu_W2'], p['stu_b2'], p['stu_a2'],
                        p['stu_W3'], p['stu_b3'])
    m_exer = _mlp_pallas(p['emb_exercise'], p['exer_W1'], p['exer_b1'],
                         p['exer_a1'], p['exer_W2'], p['exer_b2'],
                         p['exer_a2'], p['exer_W3'], p['exer_b3'])
    m_know = _mlp_pallas(p['emb_knowledge'], p['know_W1'], p['know_b1'],
                         p['know_a1'], p['know_W2'], p['know_b2'],
                         p['know_a2'], p['know_W3'], p['know_b3'])
    x1 = jnp.concatenate([m_stu, m_exer, m_know], axis=0)
    x2 = jnp.concatenate([p['stu_table'], p['exer_table'], p['know_param']],
                         axis=0)
    xcat = jnp.concatenate([x1, x2], axis=0)
    zeros = jnp.zeros((_NP, _D), jnp.float32)
    pack_r = _prep_edges(right_idx, p['right_val'])
    pack_w = _prep_edges(wrong_idx, p['wrong_val'])
    outs = _propagate(xcat, zeros, pack_r, pack_w)
    outs3 = outs.reshape(12, _NP, _D)
    xcat3 = xcat.reshape(2, _N, _D)
    wr = p['concat_W'][:, :_D]
    ww = p['concat_W'][:, _D:]
    b = p['concat_b'].reshape(1, _D)
    out1 = _combine_pallas(xcat3, outs3, 0, wr, ww, b)
    out2 = _combine_pallas(xcat3, outs3, 1, wr, ww, b)
    s1 = jnp.take(out1, student_id, axis=0)
    d1 = jnp.take(out1, _S + exercise_id, axis=0)
    s2 = jnp.take(out2, student_id, axis=0)
    d2 = jnp.take(out2, _S + exercise_id, axis=0)
    return s1, d1, s2, d2, out1[_S + _Q:], out2[_S + _Q:]
